# baseline (speedup 1.0000x reference)
import jax
import jax.numpy as jnp
from jax.experimental import pallas as pl
from jax.experimental.pallas import tpu as pltpu


def _scale_kernel(x_ref, o_ref):
    o_ref[...] = x_ref[...] * 0.9


def _half(x, half_idx):
    B, F = x.shape
    H = B // 2
    blk = 4096
    return pl.pallas_call(
        _scale_kernel,
        grid=(H // blk,),
        in_specs=[pl.BlockSpec((blk, F), lambda i, h=half_idx: (h * (H // blk) + i, 0))],
        out_specs=pl.BlockSpec((blk, F), lambda i: (i, 0)),
        out_shape=jax.ShapeDtypeStruct((H, F), x.dtype),
    )(x)


def kernel(x, W, b):
    del W, b
    return jnp.concatenate([_half(x, 0), _half(x, 1)], axis=0)


# speedup vs baseline: 2.3823x; 2.3823x over previous
import jax
import jax.numpy as jnp
from jax.experimental import pallas as pl
from jax.experimental.pallas import tpu as pltpu

_CH = 4096
_NBUF = 4


def _scale_pipeline(x_hbm, o_hbm, xbuf, obuf, in_sems, out_sems):
    B = x_hbm.shape[0]
    nch = B // _CH

    def in_copy(i, slot):
        return pltpu.make_async_copy(
            x_hbm.at[pl.ds(i * _CH, _CH), :], xbuf.at[slot], in_sems.at[slot]
        )

    def out_copy(i, slot):
        return pltpu.make_async_copy(
            obuf.at[slot], o_hbm.at[pl.ds(i * _CH, _CH), :], out_sems.at[slot]
        )

    for s in range(min(_NBUF, nch)):
        in_copy(s, s).start()
    for i in range(nch):
        s = i % _NBUF
        in_copy(i, s).wait()
        if i >= _NBUF:
            out_copy(i - _NBUF, s).wait()
        obuf[s] = xbuf[s] * 0.9
        out_copy(i, s).start()
        if i + _NBUF < nch:
            in_copy(i + _NBUF, s).start()
    for i in range(max(nch - _NBUF, 0), nch):
        out_copy(i, i % _NBUF).wait()


def kernel(x, W, b):
    del W, b
    B, F = x.shape
    return pl.pallas_call(
        _scale_pipeline,
        in_specs=[pl.BlockSpec(memory_space=pltpu.HBM)],
        out_specs=pl.BlockSpec(memory_space=pltpu.HBM),
        out_shape=jax.ShapeDtypeStruct((B, F), x.dtype),
        scratch_shapes=[
            pltpu.VMEM((_NBUF, _CH, F), x.dtype),
            pltpu.VMEM((_NBUF, _CH, F), x.dtype),
            pltpu.SemaphoreType.DMA((_NBUF,)),
            pltpu.SemaphoreType.DMA((_NBUF,)),
        ],
    )(x)
